# Initial kernel scaffold; baseline (speedup 1.0000x reference)
import jax
import jax.numpy as jnp
from jax.experimental import pallas as pl


def kernel(idx, child_tokens, child_counts, W, b_lin):
    raise NotImplementedError("write your pallas kernel here")


# speedup vs baseline: 1.9406x; 1.9406x over previous
import functools

import jax
import jax.numpy as jnp
from jax import lax
from jax.experimental import pallas as pl
from jax.experimental.pallas import tpu as pltpu
from jax.experimental.pallas import tpu_sc as plsc

B_, T_, DEPTH_, K_, VOCAB_ = 4, 64, 8, 32, 32000
ROWS = B_ * T_
PER_ROW = DEPTH_ * K_
N_UPD = ROWS * PER_ROW
LANES = 16

NUM_CORES = 2
NUM_SUBCORES = 16
NW = NUM_CORES * NUM_SUBCORES
ROWS_PER_W = ROWS // NW

_mesh = plsc.VectorSubcoreMesh(core_axis_name="c", subcore_axis_name="s")


@functools.partial(
    pl.kernel,
    out_type=jax.ShapeDtypeStruct((ROWS, VOCAB_), jnp.float32),
    mesh=_mesh,
    compiler_params=pltpu.CompilerParams(needs_layout_passes=False),
    scratch_types=[
        pltpu.VMEM((VOCAB_,), jnp.float32),
        pltpu.VMEM((PER_ROW,), jnp.int32),
        pltpu.VMEM((PER_ROW,), jnp.float32),
        pltpu.VMEM((PER_ROW,), jnp.int32),
        pltpu.VMEM((DEPTH_, LANES), jnp.float32),
        pltpu.VMEM((LANES,), jnp.float32),
    ],
)
def _scatter_rows(tok_hbm, cnt_hbm, msk_hbm, w_hbm, blin_hbm, out_hbm,
                  row_v, tok_v, cnt_v, msk_v, w_v, blin_v):
    wid = lax.axis_index("s") * NUM_CORES + lax.axis_index("c")
    pltpu.sync_copy(w_hbm, w_v)
    pltpu.sync_copy(blin_hbm, blin_v)
    blin = blin_v[...]

    def _fill(i, carry):
        row_v[pl.ds(i * LANES, LANES)] = blin
        return carry

    lax.fori_loop(0, VOCAB_ // LANES, _fill, 0)

    base = wid * ROWS_PER_W
    for i in range(ROWS_PER_W):
        r = base + i
        pltpu.sync_copy(tok_hbm.at[r], tok_v)
        pltpu.sync_copy(cnt_hbm.at[r], cnt_v)
        pltpu.sync_copy(msk_hbm.at[r], msk_v)
        for d in range(DEPTH_):
            w = w_v[d]
            for o in (d * K_, d * K_ + LANES):
                idx = tok_v[pl.ds(o, LANES)]
                m = msk_v[pl.ds(o, LANES)] != 0
                plsc.addupdate_scatter(
                    row_v, [idx], w * cnt_v[pl.ds(o, LANES)], mask=m)
        pltpu.sync_copy(row_v, out_hbm.at[r])
        for o in range(0, PER_ROW, LANES):
            plsc.store_scatter(row_v, [tok_v[pl.ds(o, LANES)]], blin)


def _round_to_bf16(x):
    u = lax.bitcast_convert_type(x, jnp.uint32)
    r = u + jnp.uint32(0x7FFF) + ((u >> 16) & jnp.uint32(1))
    return lax.bitcast_convert_type(r & jnp.uint32(0xFFFF0000), jnp.float32)


def kernel(idx, child_tokens, child_counts, W, b_lin):
    del idx
    offs = jnp.arange(ROWS * DEPTH_, dtype=jnp.int32).reshape(B_, T_, DEPTH_, 1)
    lin = (child_tokens + offs * VOCAB_).reshape(N_UPD)
    payload = lax.bitcast_convert_type(
        jnp.arange(N_UPD, dtype=jnp.int32), jnp.float32)
    skey, sval = lax.sort((lin, payload), dimension=0, is_stable=False,
                          num_keys=1)
    sperm = lax.bitcast_convert_type(sval, jnp.int32)
    win = jnp.concatenate(
        [skey[1:] != skey[:-1], jnp.ones((1,), jnp.bool_)]).astype(jnp.int32)
    mask = (jnp.zeros((N_UPD,), jnp.int32).at[sperm]
            .set(win, unique_indices=True).reshape(ROWS, PER_ROW))

    tok = child_tokens.reshape(ROWS, PER_ROW)
    cnt = _round_to_bf16(child_counts).reshape(ROWS, PER_ROW)
    w_b = jnp.broadcast_to(
        _round_to_bf16(W).reshape(DEPTH_, 1), (DEPTH_, LANES))
    blin_b = jnp.broadcast_to(b_lin.reshape(1), (LANES,))
    out = _scatter_rows(tok, cnt, mask, w_b, blin_b)
    return out.reshape(B_, T_, VOCAB_)


# speedup vs baseline: 7.6809x; 3.9580x over previous
import functools

import jax
import jax.numpy as jnp
from jax import lax
from jax.experimental import pallas as pl
from jax.experimental.pallas import tpu as pltpu
from jax.experimental.pallas import tpu_sc as plsc

B_, T_, DEPTH_, K_, VOCAB_ = 4, 64, 8, 32, 32000
ROWS = B_ * T_
PER_ROW = DEPTH_ * K_
N_UPD = ROWS * PER_ROW
LANES = 16

NUM_CORES = 2
NUM_SUBCORES = 16
NW = NUM_CORES * NUM_SUBCORES
ROWS_PER_W = ROWS // NW
CHUNK = ROWS_PER_W * PER_ROW

_mesh = plsc.VectorSubcoreMesh(core_axis_name="c", subcore_axis_name="s")


@functools.partial(
    pl.kernel,
    out_type=jax.ShapeDtypeStruct((ROWS, VOCAB_), jnp.float32),
    mesh=_mesh,
    compiler_params=pltpu.CompilerParams(needs_layout_passes=False),
    scratch_types=[
        pltpu.VMEM((VOCAB_,), jnp.float32),
        pltpu.VMEM((CHUNK,), jnp.int32),
        pltpu.VMEM((CHUNK,), jnp.int32),
        pltpu.VMEM((CHUNK,), jnp.float32),
        pltpu.VMEM((PER_ROW,), jnp.int32),
        pltpu.VMEM((LANES,), jnp.float32),
        pltpu.VMEM((LANES,), jnp.float32),
    ],
)
def _scatter_rows(key_hbm, keyn_hbm, cnt_hbm, w_hbm, blin_hbm, out_hbm,
                  row_v, key_v, keyn_v, cnt_v, tok_v, w_v, blin_v):
    wid = lax.axis_index("s") * NUM_CORES + lax.axis_index("c")
    pltpu.sync_copy(w_hbm, w_v)
    pltpu.sync_copy(blin_hbm, blin_v)
    pltpu.sync_copy(key_hbm.at[wid], key_v)
    pltpu.sync_copy(keyn_hbm.at[wid], keyn_v)
    pltpu.sync_copy(cnt_hbm.at[wid], cnt_v)
    blin = blin_v[...]

    def _fill(i, carry):
        row_v[pl.ds(i * LANES, LANES)] = blin
        return carry

    lax.fori_loop(0, VOCAB_ // LANES, _fill, 0)

    base = wid * ROWS_PER_W
    inv_v = jnp.full((LANES,), 1.0 / VOCAB_, jnp.float32)
    half_v = jnp.full((LANES,), 0.5, jnp.float32)
    for i in range(ROWS_PER_W):
        r = base + i
        rbase = jnp.broadcast_to(r * (DEPTH_ * VOCAB_), (LANES,))
        for v in range(PER_ROW // LANES):
            o = i * PER_ROW + v * LANES
            k = key_v[pl.ds(o, LANES)]
            kl = k - rbase
            d = ((kl.astype(jnp.float32) + half_v) * inv_v).astype(jnp.int32)
            tok = kl - d * VOCAB_
            tok_v[pl.ds(v * LANES, LANES)] = tok
            w = plsc.load_gather(w_v, [d])
            val = w * cnt_v[pl.ds(o, LANES)]
            win = k != keyn_v[pl.ds(o, LANES)]
            dmin = jnp.broadcast_to(jnp.min(d), (LANES,))
            is_min = d == dmin
            plsc.addupdate_scatter(row_v, [tok], val, mask=win & is_min)
            plsc.addupdate_scatter(row_v, [tok], val, mask=win & (~is_min))
        pltpu.sync_copy(row_v, out_hbm.at[r])
        for v in range(PER_ROW // LANES):
            plsc.store_scatter(row_v, [tok_v[pl.ds(v * LANES, LANES)]], blin)


def _round_to_bf16(x):
    u = lax.bitcast_convert_type(x, jnp.uint32)
    r = u + jnp.uint32(0x7FFF) + ((u >> 16) & jnp.uint32(1))
    return lax.bitcast_convert_type(r & jnp.uint32(0xFFFF0000), jnp.float32)


def kernel(idx, child_tokens, child_counts, W, b_lin):
    del idx
    offs = jnp.arange(ROWS * DEPTH_, dtype=jnp.int32).reshape(B_, T_, DEPTH_, 1)
    lin = (child_tokens + offs * VOCAB_).reshape(N_UPD)
    cnt = _round_to_bf16(child_counts).reshape(N_UPD)
    skey, scnt = lax.sort((lin, cnt), dimension=0, is_stable=False, num_keys=1)
    skey_next = jnp.concatenate([skey[1:], jnp.full((1,), -1, jnp.int32)])

    w_pad = jnp.concatenate(
        [_round_to_bf16(W).reshape(DEPTH_), jnp.zeros((DEPTH_,), jnp.float32)])
    blin_b = jnp.broadcast_to(b_lin.reshape(1), (LANES,))
    out = _scatter_rows(skey.reshape(NW, CHUNK), skey_next.reshape(NW, CHUNK),
                        scnt.reshape(NW, CHUNK), w_pad, blin_b)
    return out.reshape(B_, T_, VOCAB_)


# speedup vs baseline: 8.2359x; 1.0723x over previous
import functools

import jax
import jax.numpy as jnp
from jax import lax
from jax.experimental import pallas as pl
from jax.experimental.pallas import tpu as pltpu
from jax.experimental.pallas import tpu_sc as plsc

B_, T_, DEPTH_, K_, VOCAB_ = 4, 64, 8, 32, 32000
ROWS = B_ * T_
PER_ROW = DEPTH_ * K_
N_UPD = ROWS * PER_ROW
LANES = 16

NUM_CORES = 2
NUM_SUBCORES = 16
NW = NUM_CORES * NUM_SUBCORES
ROWS_PER_W = ROWS // NW
CHUNK = ROWS_PER_W * PER_ROW

_mesh = plsc.VectorSubcoreMesh(core_axis_name="c", subcore_axis_name="s")


@functools.partial(
    pl.kernel,
    out_type=jax.ShapeDtypeStruct((ROWS, VOCAB_), jnp.float32),
    mesh=_mesh,
    compiler_params=pltpu.CompilerParams(needs_layout_passes=False),
    scratch_types=[
        pltpu.VMEM((VOCAB_,), jnp.float32),
        pltpu.VMEM((VOCAB_,), jnp.float32),
        pltpu.VMEM((CHUNK,), jnp.int32),
        pltpu.VMEM((CHUNK,), jnp.int32),
        pltpu.VMEM((CHUNK,), jnp.float32),
        pltpu.VMEM((PER_ROW,), jnp.int32),
        pltpu.VMEM((PER_ROW,), jnp.int32),
        pltpu.VMEM((LANES,), jnp.float32),
        pltpu.VMEM((LANES,), jnp.float32),
        pltpu.SemaphoreType.DMA,
        pltpu.SemaphoreType.DMA,
        pltpu.SemaphoreType.DMA,
    ],
)
def _scatter_rows(key_hbm, keyn_hbm, cnt_hbm, w_hbm, blin_hbm, out_hbm,
                  row_a, row_b, key_v, keyn_v, cnt_v, tok_a, tok_b,
                  w_v, blin_v, sem0, sem1, sem_in):
    wid = lax.axis_index("s") * NUM_CORES + lax.axis_index("c")
    pltpu.sync_copy(w_hbm, w_v)
    pltpu.sync_copy(blin_hbm, blin_v)
    in0 = pltpu.async_copy(key_hbm.at[wid], key_v, sem_in)
    in1 = pltpu.async_copy(keyn_hbm.at[wid], keyn_v, sem_in)
    in2 = pltpu.async_copy(cnt_hbm.at[wid], cnt_v, sem_in)
    blin = blin_v[...]

    def _fill(i, carry):
        row_a[pl.ds(i * LANES, LANES)] = blin
        row_b[pl.ds(i * LANES, LANES)] = blin
        return carry

    lax.fori_loop(0, VOCAB_ // LANES, _fill, 0)
    in0.wait()
    in1.wait()
    in2.wait()

    base = wid * ROWS_PER_W
    inv_v = jnp.full((LANES,), 1.0 / VOCAB_, jnp.float32)
    half_v = jnp.full((LANES,), 0.5, jnp.float32)
    sems = (sem0, sem1)
    rows = (row_a, row_b)
    toks = (tok_a, tok_b)
    out_dma = [None, None]
    for i in range(ROWS_PER_W):
        bi = i % 2
        r = base + i
        row_v, tok_v = rows[bi], toks[bi]
        if out_dma[bi] is not None:
            out_dma[bi].wait()
            for v in range(PER_ROW // LANES):
                plsc.store_scatter(
                    row_v, [tok_v[pl.ds(v * LANES, LANES)]], blin)
        rbase = jnp.broadcast_to(r * (DEPTH_ * VOCAB_), (LANES,))
        for v in range(PER_ROW // LANES):
            o = i * PER_ROW + v * LANES
            k = key_v[pl.ds(o, LANES)]
            kl = k - rbase
            d = ((kl.astype(jnp.float32) + half_v) * inv_v).astype(jnp.int32)
            tok = kl - d * VOCAB_
            tok_v[pl.ds(v * LANES, LANES)] = tok
            w = plsc.load_gather(w_v, [d])
            val = w * cnt_v[pl.ds(o, LANES)]
            win = k != keyn_v[pl.ds(o, LANES)]
            dmin = jnp.broadcast_to(jnp.min(d), (LANES,))
            is_min = d == dmin
            plsc.addupdate_scatter(row_v, [tok], val, mask=win & is_min)
            plsc.addupdate_scatter(row_v, [tok], val, mask=win & (~is_min))
        out_dma[bi] = pltpu.async_copy(row_v, out_hbm.at[r], sems[bi])
    out_dma[0].wait()
    out_dma[1].wait()


def _round_to_bf16(x):
    u = lax.bitcast_convert_type(x, jnp.uint32)
    r = u + jnp.uint32(0x7FFF) + ((u >> 16) & jnp.uint32(1))
    return lax.bitcast_convert_type(r & jnp.uint32(0xFFFF0000), jnp.float32)


def kernel(idx, child_tokens, child_counts, W, b_lin):
    del idx
    offs = jnp.arange(ROWS * DEPTH_, dtype=jnp.int32).reshape(B_, T_, DEPTH_, 1)
    lin = (child_tokens + offs * VOCAB_).reshape(N_UPD)
    cnt = _round_to_bf16(child_counts).reshape(N_UPD)
    skey, scnt = lax.sort((lin, cnt), dimension=0, is_stable=False, num_keys=1)
    skey_next = jnp.concatenate([skey[1:], jnp.full((1,), -1, jnp.int32)])

    w_pad = jnp.concatenate(
        [_round_to_bf16(W).reshape(DEPTH_), jnp.zeros((DEPTH_,), jnp.float32)])
    blin_b = jnp.broadcast_to(b_lin.reshape(1), (LANES,))
    out = _scatter_rows(skey.reshape(NW, CHUNK), skey_next.reshape(NW, CHUNK),
                        scnt.reshape(NW, CHUNK), w_pad, blin_b)
    return out.reshape(B_, T_, VOCAB_)


# speedup vs baseline: 9.1278x; 1.1083x over previous
import functools

import jax
import jax.numpy as jnp
from jax import lax
from jax.experimental import pallas as pl
from jax.experimental.pallas import tpu as pltpu
from jax.experimental.pallas import tpu_sc as plsc

B_, T_, DEPTH_, K_, VOCAB_ = 4, 64, 8, 32, 32000
ROWS = B_ * T_
PER_ROW = DEPTH_ * K_
N_UPD = ROWS * PER_ROW
LANES = 16

NUM_CORES = 2
NUM_SUBCORES = 16
NW = NUM_CORES * NUM_SUBCORES
ROWS_PER_W = ROWS // NW
CHUNK = ROWS_PER_W * PER_ROW

_mesh = plsc.VectorSubcoreMesh(core_axis_name="c", subcore_axis_name="s")


@functools.partial(
    pl.kernel,
    out_type=jax.ShapeDtypeStruct((ROWS, VOCAB_), jnp.float32),
    mesh=_mesh,
    compiler_params=pltpu.CompilerParams(needs_layout_passes=False),
    scratch_types=[
        pltpu.VMEM((VOCAB_,), jnp.float32),
        pltpu.VMEM((VOCAB_,), jnp.float32),
        pltpu.VMEM((CHUNK,), jnp.int32),
        pltpu.VMEM((CHUNK,), jnp.int32),
        pltpu.VMEM((CHUNK,), jnp.float32),
        pltpu.VMEM((PER_ROW,), jnp.int32),
        pltpu.VMEM((PER_ROW,), jnp.int32),
        pltpu.VMEM((LANES,), jnp.float32),
        pltpu.VMEM((LANES,), jnp.float32),
        pltpu.SemaphoreType.DMA,
        pltpu.SemaphoreType.DMA,
        pltpu.SemaphoreType.DMA,
    ],
)
def _scatter_rows(key_hbm, keyn_hbm, cnt_hbm, w_hbm, blin_hbm, out_hbm,
                  row_a, row_b, key_v, keyn_v, cnt_v, tok_a, tok_b,
                  w_v, blin_v, sem0, sem1, sem_in):
    wid = lax.axis_index("s") * NUM_CORES + lax.axis_index("c")
    pltpu.sync_copy(w_hbm, w_v)
    pltpu.sync_copy(blin_hbm, blin_v)
    in0 = pltpu.async_copy(key_hbm.at[wid], key_v, sem_in)
    in1 = pltpu.async_copy(keyn_hbm.at[wid], keyn_v, sem_in)
    in2 = pltpu.async_copy(cnt_hbm.at[wid], cnt_v, sem_in)
    blin = blin_v[...]

    FILL_UNROLL = 8

    def _fill(i, carry):
        for u in range(FILL_UNROLL):
            row_a[pl.ds((i * FILL_UNROLL + u) * LANES, LANES)] = blin
            row_b[pl.ds((i * FILL_UNROLL + u) * LANES, LANES)] = blin
        return carry

    lax.fori_loop(0, VOCAB_ // (LANES * FILL_UNROLL), _fill, 0)
    in0.wait()
    in1.wait()
    in2.wait()

    base = wid * ROWS_PER_W
    inv_v = jnp.full((LANES,), 1.0 / VOCAB_, jnp.float32)
    half_v = jnp.full((LANES,), 0.5, jnp.float32)
    sems = (sem0, sem1)
    rows = (row_a, row_b)
    toks = (tok_a, tok_b)
    out_dma = [None, None]
    for i in range(ROWS_PER_W):
        bi = i % 2
        r = base + i
        row_v, tok_v = rows[bi], toks[bi]
        if out_dma[bi] is not None:
            out_dma[bi].wait()
            for v in range(PER_ROW // LANES):
                plsc.store_scatter(
                    row_v, [tok_v[pl.ds(v * LANES, LANES)]], blin)
        rbase = jnp.broadcast_to(r * (DEPTH_ * VOCAB_), (LANES,))
        for v in range(PER_ROW // LANES):
            o = i * PER_ROW + v * LANES
            k = key_v[pl.ds(o, LANES)]
            kl = k - rbase
            d = ((kl.astype(jnp.float32) + half_v) * inv_v).astype(jnp.int32)
            tok = kl - d * VOCAB_
            tok_v[pl.ds(v * LANES, LANES)] = tok
            w = plsc.load_gather(w_v, [d])
            val = w * cnt_v[pl.ds(o, LANES)]
            win = k != keyn_v[pl.ds(o, LANES)]
            even = (d & 1) == 0
            plsc.addupdate_scatter(row_v, [tok], val, mask=win & even)
            plsc.addupdate_scatter(row_v, [tok], val, mask=win & (~even))
        out_dma[bi] = pltpu.async_copy(row_v, out_hbm.at[r], sems[bi])
    out_dma[0].wait()
    out_dma[1].wait()


def _round_to_bf16(x):
    u = lax.bitcast_convert_type(x, jnp.uint32)
    r = u + jnp.uint32(0x7FFF) + ((u >> 16) & jnp.uint32(1))
    return lax.bitcast_convert_type(r & jnp.uint32(0xFFFF0000), jnp.float32)


def kernel(idx, child_tokens, child_counts, W, b_lin):
    del idx
    offs = jnp.arange(ROWS * DEPTH_, dtype=jnp.int32).reshape(B_, T_, DEPTH_, 1)
    lin = (child_tokens + offs * VOCAB_).reshape(N_UPD)
    cnt = _round_to_bf16(child_counts).reshape(N_UPD)
    skey, scnt = lax.sort((lin, cnt), dimension=0, is_stable=False, num_keys=1)
    skey_next = jnp.concatenate([skey[1:], jnp.full((1,), -1, jnp.int32)])

    w_pad = jnp.concatenate(
        [_round_to_bf16(W).reshape(DEPTH_), jnp.zeros((DEPTH_,), jnp.float32)])
    blin_b = jnp.broadcast_to(b_lin.reshape(1), (LANES,))
    out = _scatter_rows(skey.reshape(NW, CHUNK), skey_next.reshape(NW, CHUNK),
                        scnt.reshape(NW, CHUNK), w_pad, blin_b)
    return out.reshape(B_, T_, VOCAB_)


# speedup vs baseline: 9.2505x; 1.0134x over previous
import functools

import jax
import jax.numpy as jnp
from jax import lax
from jax.experimental import pallas as pl
from jax.experimental.pallas import tpu as pltpu
from jax.experimental.pallas import tpu_sc as plsc

B_, T_, DEPTH_, K_, VOCAB_ = 4, 64, 8, 32, 32000
ROWS = B_ * T_
PER_ROW = DEPTH_ * K_
N_UPD = ROWS * PER_ROW
LANES = 16

NUM_CORES = 2
NUM_SUBCORES = 16
NW = NUM_CORES * NUM_SUBCORES
ROWS_PER_W = ROWS // NW
CHUNK = ROWS_PER_W * PER_ROW

_mesh = plsc.VectorSubcoreMesh(core_axis_name="c", subcore_axis_name="s")


@functools.partial(
    pl.kernel,
    out_type=jax.ShapeDtypeStruct((ROWS, VOCAB_), jnp.float32),
    mesh=_mesh,
    compiler_params=pltpu.CompilerParams(needs_layout_passes=False),
    scratch_types=[
        pltpu.VMEM((VOCAB_,), jnp.float32),
        pltpu.VMEM((VOCAB_,), jnp.float32),
        pltpu.VMEM((CHUNK + LANES,), jnp.int32),
        pltpu.VMEM((CHUNK,), jnp.float32),
        pltpu.VMEM((PER_ROW,), jnp.int32),
        pltpu.VMEM((PER_ROW,), jnp.int32),
        pltpu.VMEM((LANES,), jnp.float32),
        pltpu.VMEM((LANES,), jnp.float32),
        pltpu.SemaphoreType.DMA,
        pltpu.SemaphoreType.DMA,
        pltpu.SemaphoreType.DMA,
    ],
)
def _scatter_rows(key_hbm, cnt_hbm, w_hbm, blin_hbm, out_hbm,
                  row_a, row_b, key_v, cnt_v, tok_a, tok_b,
                  w_v, blin_v, sem0, sem1, sem_in):
    wid = lax.axis_index("s") * NUM_CORES + lax.axis_index("c")
    pltpu.sync_copy(w_hbm, w_v)
    pltpu.sync_copy(blin_hbm, blin_v)
    in0 = pltpu.async_copy(key_hbm.at[wid], key_v.at[pl.ds(0, CHUNK)], sem_in)
    in2 = pltpu.async_copy(cnt_hbm.at[wid], cnt_v, sem_in)
    key_v[pl.ds(CHUNK, LANES)] = jnp.full((LANES,), -1, jnp.int32)
    blin = blin_v[...]

    FILL_UNROLL = 8

    def _fill(i, carry):
        for u in range(FILL_UNROLL):
            row_a[pl.ds((i * FILL_UNROLL + u) * LANES, LANES)] = blin
            row_b[pl.ds((i * FILL_UNROLL + u) * LANES, LANES)] = blin
        return carry

    lax.fori_loop(0, VOCAB_ // (LANES * FILL_UNROLL), _fill, 0)
    in0.wait()
    in2.wait()

    base = wid * ROWS_PER_W
    inv_v = jnp.full((LANES,), 1.0 / VOCAB_, jnp.float32)
    half_v = jnp.full((LANES,), 0.5, jnp.float32)
    sems = (sem0, sem1)
    rows = (row_a, row_b)
    toks = (tok_a, tok_b)
    out_dma = [None, None]
    for i in range(ROWS_PER_W):
        bi = i % 2
        r = base + i
        row_v, tok_v = rows[bi], toks[bi]
        if out_dma[bi] is not None:
            out_dma[bi].wait()
            for v in range(PER_ROW // LANES):
                plsc.store_scatter(
                    row_v, [tok_v[pl.ds(v * LANES, LANES)]], blin)
        rbase = jnp.broadcast_to(r * (DEPTH_ * VOCAB_), (LANES,))
        for v in range(PER_ROW // LANES):
            o = i * PER_ROW + v * LANES
            k = key_v[pl.ds(o, LANES)]
            kl = k - rbase
            d = ((kl.astype(jnp.float32) + half_v) * inv_v).astype(jnp.int32)
            tok = kl - d * VOCAB_
            tok_v[pl.ds(v * LANES, LANES)] = tok
            w = plsc.load_gather(w_v, [d])
            val = w * cnt_v[pl.ds(o, LANES)]
            win = k != key_v[pl.ds(o + 1, LANES)]
            even = (d & 1) == 0
            plsc.addupdate_scatter(row_v, [tok], val, mask=win & even)
            plsc.addupdate_scatter(row_v, [tok], val, mask=win & (~even))
        out_dma[bi] = pltpu.async_copy(row_v, out_hbm.at[r], sems[bi])
    out_dma[0].wait()
    out_dma[1].wait()


def _round_to_bf16(x):
    u = lax.bitcast_convert_type(x, jnp.uint32)
    r = u + jnp.uint32(0x7FFF) + ((u >> 16) & jnp.uint32(1))
    return lax.bitcast_convert_type(r & jnp.uint32(0xFFFF0000), jnp.float32)


def kernel(idx, child_tokens, child_counts, W, b_lin):
    del idx
    offs = jnp.arange(ROWS * DEPTH_, dtype=jnp.int32).reshape(B_, T_, DEPTH_, 1)
    lin = (child_tokens + offs * VOCAB_).reshape(N_UPD)
    cnt = _round_to_bf16(child_counts).reshape(N_UPD)
    skey, scnt = lax.sort((lin, cnt), dimension=0, is_stable=False, num_keys=1)

    w_pad = jnp.concatenate(
        [_round_to_bf16(W).reshape(DEPTH_), jnp.zeros((DEPTH_,), jnp.float32)])
    blin_b = jnp.broadcast_to(b_lin.reshape(1), (LANES,))
    out = _scatter_rows(skey.reshape(NW, CHUNK),
                        scnt.reshape(NW, CHUNK), w_pad, blin_b)
    return out.reshape(B_, T_, VOCAB_)
